# baseline (speedup 1.0000x reference)
import jax
import jax.numpy as jnp
from jax.experimental import pallas as pl
from jax.experimental.pallas import tpu as pltpu

_EPS = 1e-5
_MIN_NORM = 1e-15
_MAXNORM = 1.0 - _EPS


def _artanh(x):
    x = jnp.clip(x, -1 + 1e-7, 1 - 1e-7)
    return 0.5 * jnp.log((1 + x) / (1 - x))


def _rnorm2(x):
    return jnp.sum(x * x, axis=-1, keepdims=True)


def _clipn(n):
    return jnp.clip(n, _MIN_NORM, None)


_BM = 512
_NBUF = 4
_LOOK = 3


def _hgc_kernel(adj_ref, x_ref, w_ref, b_ref, bo_ref, out_ref, xt_ref, abuf, sem, xbuf, xsem):
    i = pl.program_id(0)
    nblk = adj_ref.shape[0] // _BM

    def _copy(blk, slot):
        return pltpu.make_async_copy(
            adj_ref.at[pl.ds(blk * _BM, _BM), :], abuf.at[slot], sem.at[slot]
        )

    @pl.when(i == 0)
    def _stage1():
        xcopy = pltpu.make_async_copy(x_ref, xbuf, xsem)
        xcopy.start()
        for b0 in range(_LOOK):
            _copy(b0, b0).start()
        xcopy.wait()
        x = xbuf[...]
        w = w_ref[...]
        b = b_ref[...]
        n_b = _clipn(jnp.sqrt(_rnorm2(b)))
        eb = jnp.tanh(n_b) * b / n_b
        n_eb = _clipn(jnp.sqrt(_rnorm2(eb)))
        hb = jnp.where(n_eb > _MAXNORM, eb / n_eb * _MAXNORM, eb)
        y2 = _rnorm2(hb)
        v = jnp.dot(hb, w, preferred_element_type=jnp.float32)
        n_x = _clipn(jnp.sqrt(_rnorm2(x)))
        xy0 = jnp.sum(x * v, axis=-1, keepdims=True)
        mx = jax.lax.dot_general(
            x, w, (((1,), (1,)), ((), ())), preferred_element_type=jnp.float32
        )
        n_mx = _clipn(jnp.sqrt(_rnorm2(mx)))
        t1 = jnp.tanh(n_mx / n_x * _artanh(n_x))
        s1 = t1 / n_mx
        s2 = jnp.where(t1 > _MAXNORM, _MAXNORM / t1, 1.0)
        sr = s1 * s2
        r = jnp.minimum(t1, _MAXNORM)
        xy = xy0 * sr
        x2 = r * r
        alpha = 1 + 2 * xy + y2
        beta = 1 - x2
        den = _clipn(1 + 2 * xy + x2 * y2)
        a0 = alpha / den
        b0 = beta / den
        ma_n2 = a0 * a0 * x2 + 2 * a0 * b0 * xy + b0 * b0 * y2
        n_ma = _clipn(jnp.sqrt(ma_n2))
        s3 = jnp.where(n_ma > _MAXNORM, _MAXNORM / n_ma, 1.0)
        n2 = _clipn(jnp.minimum(n_ma, _MAXNORM))
        sc = s3 * _artanh(n2) / n2
        aa = a0 * sr * sc
        bb = b0 * sc
        xt_ref[...] = (aa * mx + bb * hb).astype(jnp.bfloat16)

    @pl.when(i > 0)
    def _stage2():
        j = i - 1
        slot = jax.lax.rem(j, _NBUF)
        _copy(j, slot).wait()
        s = jnp.dot(abuf[slot], xt_ref[...], preferred_element_type=jnp.float32)

        @pl.when(j + _LOOK < nblk)
        def _prefetch():
            nslot = jax.lax.rem(j + _LOOK, _NBUF)
            _copy(j + _LOOK, nslot).start()

        n = _clipn(jnp.sqrt(_rnorm2(s)))
        t = jnp.tanh(n)
        f = jnp.where(t > _MAXNORM, _MAXNORM / n, t / n)
        out_ref[...] = s * f + bo_ref[...]


def kernel(adjacency, input_feature, W, b_lin, bias_out):
    N, din = input_feature.shape
    dout = W.shape[0]
    nblk = N // _BM
    b2 = b_lin.reshape(1, dout).astype(jnp.float32)
    bo2 = bias_out.reshape(1, dout).astype(jnp.float32)
    return pl.pallas_call(
        _hgc_kernel,
        grid=(nblk + 1,),
        in_specs=[
            pl.BlockSpec(memory_space=pl.ANY),
            pl.BlockSpec(memory_space=pl.ANY),
            pl.BlockSpec((dout, din), lambda i: (0, 0)),
            pl.BlockSpec((1, dout), lambda i: (0, 0)),
            pl.BlockSpec((1, dout), lambda i: (0, 0)),
        ],
        out_specs=pl.BlockSpec((_BM, dout), lambda i: (jnp.maximum(i - 1, 0), 0)),
        out_shape=jax.ShapeDtypeStruct((N, dout), jnp.float32),
        scratch_shapes=[
            pltpu.VMEM((N, dout), jnp.bfloat16),
            pltpu.VMEM((_NBUF, _BM, N), jnp.float32),
            pltpu.SemaphoreType.DMA((_NBUF,)),
            pltpu.VMEM((N, din), jnp.float32),
            pltpu.SemaphoreType.DMA,
        ],
    )(adjacency, input_feature, W, b2, bo2)


# speedup vs baseline: 1.1680x; 1.1680x over previous
import jax
import jax.numpy as jnp
from jax.experimental import pallas as pl
from jax.experimental.pallas import tpu as pltpu

_EPS = 1e-5
_MIN_NORM = 1e-15
_MAXNORM = 1.0 - _EPS


def _artanh(x):
    x = jnp.clip(x, -1 + 1e-7, 1 - 1e-7)
    return 0.5 * jnp.log((1 + x) / (1 - x))


def _rnorm2(x):
    return jnp.sum(x * x, axis=-1, keepdims=True)


def _clipn(n):
    return jnp.clip(n, _MIN_NORM, None)


_BM = 512
_NBUF = 4
_LOOK = 3


def _hgc_kernel(adj_ref, x_ref, w_ref, b_ref, bo_ref, out_ref, xt_ref, abuf, sem):
    i = pl.program_id(0)
    nblk = adj_ref.shape[0] // _BM

    def _copy(blk, slot):
        return pltpu.make_async_copy(
            adj_ref.at[pl.ds(blk * _BM, _BM), :], abuf.at[slot], sem.at[slot]
        )

    @pl.when(i == 0)
    def _stage1():
        for b0 in range(_LOOK):
            _copy(b0, b0).start()
        x = x_ref[...]
        w = w_ref[...]
        b = b_ref[...]
        n_b = _clipn(jnp.sqrt(_rnorm2(b)))
        eb = jnp.tanh(n_b) * b / n_b
        n_eb = _clipn(jnp.sqrt(_rnorm2(eb)))
        hb = jnp.where(n_eb > _MAXNORM, eb / n_eb * _MAXNORM, eb)
        y2 = _rnorm2(hb)
        v = jnp.dot(hb, w, preferred_element_type=jnp.float32)
        n_x = _clipn(jnp.sqrt(_rnorm2(x)))
        xy0 = jnp.sum(x * v, axis=-1, keepdims=True)
        mx = jax.lax.dot_general(
            x, w, (((1,), (1,)), ((), ())), preferred_element_type=jnp.float32
        )
        n_mx = _clipn(jnp.sqrt(_rnorm2(mx)))
        t1 = jnp.tanh(n_mx / n_x * _artanh(n_x))
        s1 = t1 / n_mx
        s2 = jnp.where(t1 > _MAXNORM, _MAXNORM / t1, 1.0)
        sr = s1 * s2
        r = jnp.minimum(t1, _MAXNORM)
        xy = xy0 * sr
        x2 = r * r
        alpha = 1 + 2 * xy + y2
        beta = 1 - x2
        den = _clipn(1 + 2 * xy + x2 * y2)
        a0 = alpha / den
        b0 = beta / den
        ma_n2 = a0 * a0 * x2 + 2 * a0 * b0 * xy + b0 * b0 * y2
        n_ma = _clipn(jnp.sqrt(ma_n2))
        s3 = jnp.where(n_ma > _MAXNORM, _MAXNORM / n_ma, 1.0)
        n2 = _clipn(jnp.minimum(n_ma, _MAXNORM))
        sc = s3 * _artanh(n2) / n2
        aa = a0 * sr * sc
        bb = b0 * sc
        xt_ref[...] = (aa * mx + bb * hb).astype(jnp.bfloat16)

    @pl.when(i > 0)
    def _stage2():
        j = i - 1
        slot = jax.lax.rem(j, _NBUF)
        _copy(j, slot).wait()
        s = jnp.dot(abuf[slot], xt_ref[...], preferred_element_type=jnp.float32)

        @pl.when(j + _LOOK < nblk)
        def _prefetch():
            nslot = jax.lax.rem(j + _LOOK, _NBUF)
            _copy(j + _LOOK, nslot).start()

        n = _clipn(jnp.sqrt(_rnorm2(s)))
        t = jnp.tanh(n)
        f = jnp.where(t > _MAXNORM, _MAXNORM / n, t / n)
        out_ref[...] = s * f + bo_ref[...]


def kernel(adjacency, input_feature, W, b_lin, bias_out):
    N, din = input_feature.shape
    dout = W.shape[0]
    nblk = N // _BM
    b2 = b_lin.reshape(1, dout).astype(jnp.float32)
    bo2 = bias_out.reshape(1, dout).astype(jnp.float32)
    return pl.pallas_call(
        _hgc_kernel,
        grid=(nblk + 1,),
        in_specs=[
            pl.BlockSpec(memory_space=pl.ANY),
            pl.BlockSpec((N, din), lambda i: (0, 0)),
            pl.BlockSpec((dout, din), lambda i: (0, 0)),
            pl.BlockSpec((1, dout), lambda i: (0, 0)),
            pl.BlockSpec((1, dout), lambda i: (0, 0)),
        ],
        out_specs=pl.BlockSpec((_BM, dout), lambda i: (jnp.maximum(i - 1, 0), 0)),
        out_shape=jax.ShapeDtypeStruct((N, dout), jnp.float32),
        scratch_shapes=[
            pltpu.VMEM((N, dout), jnp.bfloat16),
            pltpu.VMEM((_NBUF, _BM, N), jnp.float32),
            pltpu.SemaphoreType.DMA((_NBUF,)),
        ],
    )(adjacency, input_feature, W, b2, bo2)


# speedup vs baseline: 1.1690x; 1.0009x over previous
import jax
import jax.numpy as jnp
from jax.experimental import pallas as pl
from jax.experimental.pallas import tpu as pltpu

_EPS = 1e-5
_MIN_NORM = 1e-15
_MAXNORM = 1.0 - _EPS


def _artanh(x):
    x = jnp.clip(x, -1 + 1e-7, 1 - 1e-7)
    return 0.5 * jnp.log((1 + x) / (1 - x))


def _rnorm2(x):
    return jnp.sum(x * x, axis=-1, keepdims=True)


def _clipn(n):
    return jnp.clip(n, _MIN_NORM, None)


_BM = 512
_S1CHUNKS = 4
_NBUF = 4
_LOOK = 3


def _hgc_kernel(adj_ref, x_ref, w_ref, b_ref, bo_ref, out_ref, xt_ref, abuf, sem):
    i = pl.program_id(0)
    nblk = adj_ref.shape[0] // _BM

    def _copy(blk, slot):
        return pltpu.make_async_copy(
            adj_ref.at[pl.ds(blk * _BM, _BM), :], abuf.at[slot], sem.at[slot]
        )

    @pl.when(i == 0)
    def _stage1():
        for b0 in range(_LOOK):
            _copy(b0, b0).start()
        w = w_ref[...]
        b = b_ref[...]
        n_b = _clipn(jnp.sqrt(_rnorm2(b)))
        eb = jnp.tanh(n_b) * b / n_b
        n_eb = _clipn(jnp.sqrt(_rnorm2(eb)))
        hb = jnp.where(n_eb > _MAXNORM, eb / n_eb * _MAXNORM, eb)
        y2 = _rnorm2(hb)
        v = jnp.dot(hb, w, preferred_element_type=jnp.float32)
        n_rows = x_ref.shape[0]
        cs = n_rows // _S1CHUNKS
        for h in range(_S1CHUNKS):
            x = x_ref[h * cs : (h + 1) * cs, :]
            n_x = _clipn(jnp.sqrt(_rnorm2(x)))
            xy0 = jnp.sum(x * v, axis=-1, keepdims=True)
            mx = jax.lax.dot_general(
                x, w, (((1,), (1,)), ((), ())), preferred_element_type=jnp.float32
            )
            n_mx = _clipn(jnp.sqrt(_rnorm2(mx)))
            t1 = jnp.tanh(n_mx / n_x * _artanh(n_x))
            s1 = t1 / n_mx
            s2 = jnp.where(t1 > _MAXNORM, _MAXNORM / t1, 1.0)
            sr = s1 * s2
            r = jnp.minimum(t1, _MAXNORM)
            xy = xy0 * sr
            x2 = r * r
            alpha = 1 + 2 * xy + y2
            beta = 1 - x2
            den = _clipn(1 + 2 * xy + x2 * y2)
            a0 = alpha / den
            b0 = beta / den
            ma_n2 = a0 * a0 * x2 + 2 * a0 * b0 * xy + b0 * b0 * y2
            n_ma = _clipn(jnp.sqrt(ma_n2))
            s3 = jnp.where(n_ma > _MAXNORM, _MAXNORM / n_ma, 1.0)
            n2 = _clipn(jnp.minimum(n_ma, _MAXNORM))
            sc = s3 * _artanh(n2) / n2
            aa = a0 * sr * sc
            bb = b0 * sc
            xt_ref[h * cs : (h + 1) * cs, :] = (aa * mx + bb * hb).astype(
                jnp.bfloat16
            )

    @pl.when(i > 0)
    def _stage2():
        j = i - 1
        slot = jax.lax.rem(j, _NBUF)
        _copy(j, slot).wait()
        s = jnp.dot(abuf[slot], xt_ref[...], preferred_element_type=jnp.float32)

        @pl.when(j + _LOOK < nblk)
        def _prefetch():
            nslot = jax.lax.rem(j + _LOOK, _NBUF)
            _copy(j + _LOOK, nslot).start()

        n = _clipn(jnp.sqrt(_rnorm2(s)))
        t = jnp.tanh(n)
        f = jnp.where(t > _MAXNORM, _MAXNORM / n, t / n)
        out_ref[...] = s * f + bo_ref[...]


def kernel(adjacency, input_feature, W, b_lin, bias_out):
    N, din = input_feature.shape
    dout = W.shape[0]
    nblk = N // _BM
    b2 = b_lin.reshape(1, dout).astype(jnp.float32)
    bo2 = bias_out.reshape(1, dout).astype(jnp.float32)
    return pl.pallas_call(
        _hgc_kernel,
        grid=(nblk + 1,),
        in_specs=[
            pl.BlockSpec(memory_space=pl.ANY),
            pl.BlockSpec((N, din), lambda i: (0, 0)),
            pl.BlockSpec((dout, din), lambda i: (0, 0)),
            pl.BlockSpec((1, dout), lambda i: (0, 0)),
            pl.BlockSpec((1, dout), lambda i: (0, 0)),
        ],
        out_specs=pl.BlockSpec((_BM, dout), lambda i: (jnp.maximum(i - 1, 0), 0)),
        out_shape=jax.ShapeDtypeStruct((N, dout), jnp.float32),
        scratch_shapes=[
            pltpu.VMEM((N, dout), jnp.bfloat16),
            pltpu.VMEM((_NBUF, _BM, N), jnp.float32),
            pltpu.SemaphoreType.DMA((_NBUF,)),
        ],
    )(adjacency, input_feature, W, b2, bo2)
